# Initial kernel scaffold; baseline (speedup 1.0000x reference)
import jax
import jax.numpy as jnp
from jax.experimental import pallas as pl


def kernel(indices, table, W, b):
    raise NotImplementedError("write your pallas kernel here")


# speedup vs baseline: 6.4272x; 6.4272x over previous
import functools

import jax
import jax.numpy as jnp
from jax import lax
from jax.experimental import pallas as pl
from jax.experimental.pallas import tpu as pltpu
from jax.experimental.pallas import tpu_sc as plsc

NUM_EMB = 1000001
EMB_DIM = 16
BATCH = 16384
SEQ = 200

NC = 2
NS = 16
NW = NC * NS
RPW = BATCH // NW
L_CHUNK = 100
N_CHUNKS = SEQ // L_CHUNK


def _project_body(tbl_ref, w_ref, out_ref):
    acc = jnp.dot(tbl_ref[...], w_ref[...], preferred_element_type=jnp.float32)
    out_ref[...] = acc * (1.0 / SEQ)


def _project(table, W):
    BR = 8192
    grid = (NUM_EMB + BR - 1) // BR
    return pl.pallas_call(
        _project_body,
        grid=(grid,),
        in_specs=[
            pl.BlockSpec((BR, EMB_DIM), lambda i: (i, 0)),
            pl.BlockSpec((EMB_DIM, 1), lambda i: (0, 0)),
        ],
        out_specs=pl.BlockSpec((BR, 1), lambda i: (i, 0)),
        out_shape=jax.ShapeDtypeStruct((NUM_EMB, 1), jnp.float32),
    )(table, W)


def _pool_body(idx_hbm, tv_hbm, b_hbm, out_hbm, idx_v, vals_v, bias_v, acc_v, sem):
    w = lax.axis_index("s") * NC + lax.axis_index("c")

    pltpu.sync_copy(b_hbm, bias_v)
    bias = bias_v[...]

    zero = jnp.zeros((16,), jnp.float32)
    for g in range(RPW // 16):
        acc_v[pl.ds(16 * g, 16)] = zero

    for c in range(N_CHUNKS):
        pltpu.sync_copy(idx_hbm.at[w, pl.ds(c * L_CHUNK * RPW, L_CHUNK * RPW)],
                        idx_v)
        pltpu.async_copy(tv_hbm.at[idx_v], vals_v, sem).wait()
        for g in range(RPW // 16):
            def body(l, a):
                return a + vals_v[pl.ds(l * RPW + 16 * g, 16)]
            part = lax.fori_loop(0, L_CHUNK, body, zero)
            acc_v[pl.ds(16 * g, 16)] += part

    for g in range(RPW // 16):
        acc_v[pl.ds(16 * g, 16)] += bias
    pltpu.sync_copy(acc_v, out_hbm.at[pl.ds(w * RPW, RPW)])


def _pool(idx3d, tv_flat, b16):
    mesh = plsc.VectorSubcoreMesh(core_axis_name="c", subcore_axis_name="s")
    f = pl.kernel(
        _pool_body,
        out_type=jax.ShapeDtypeStruct((BATCH,), jnp.float32),
        mesh=mesh,
        scratch_types=[
            pltpu.VMEM((L_CHUNK * RPW,), jnp.int32),
            pltpu.VMEM((L_CHUNK * RPW,), jnp.float32),
            pltpu.VMEM((16,), jnp.float32),
            pltpu.VMEM((RPW,), jnp.float32),
            pltpu.SemaphoreType.DMA,
        ],
    )
    return f(idx3d, tv_flat, b16)


def kernel(indices, table, W, b):
    tv = _project(table, W)
    tv_flat = tv.reshape(NUM_EMB)
    idx = indices.astype(jnp.int32)
    idx3d = (idx.reshape(NW, RPW, SEQ)
                .transpose(0, 2, 1)
                .reshape(NW, SEQ * RPW))
    b16 = jnp.broadcast_to(b.astype(jnp.float32), (16,))
    out = _pool(idx3d, tv_flat, b16)
    return out.reshape(BATCH, 1)


# speedup vs baseline: 8.8361x; 1.3748x over previous
import functools

import jax
import jax.numpy as jnp
from jax import lax
from jax.experimental import pallas as pl
from jax.experimental.pallas import tpu as pltpu
from jax.experimental.pallas import tpu_sc as plsc

NUM_EMB = 1000001
EMB_DIM = 16
BATCH = 16384
SEQ = 200

NC = 2
NS = 16
NW = NC * NS
RPW = BATCH // NW
L_CHUNK = 100
N_CHUNKS = SEQ // L_CHUNK


F_ROWS = 125000
QB = 608
NBLK = 13
FB = 16 * QB
LAST_Q, LAST_Y = divmod(1000000, 128)


def _project_body(f_ref, t_ref, last_ref, w_ref, out_ref):
    b = pl.program_id(0)
    f = f_ref[...]
    frows = lax.broadcasted_iota(jnp.int32, (FB, 128), 0)
    f = jnp.where(frows < F_ROWS - FB * b, f, 0.0)
    g = f.reshape(QB, 16, 128)
    acc = jnp.zeros((QB, 128), jnp.float32)
    for j in range(16):
        acc += jnp.dot(g[:, j, :], t_ref[j],
                       preferred_element_type=jnp.float32)

    @pl.when(b == LAST_Q // QB)
    def _():
        s = jnp.dot(last_ref[...], w_ref[...],
                    preferred_element_type=jnp.float32)[0, 0] * (1.0 / SEQ)
        rows = lax.broadcasted_iota(jnp.int32, (QB, 128), 0)
        cols = lax.broadcasted_iota(jnp.int32, (QB, 128), 1)
        mask = (rows == LAST_Q % QB) & (cols == LAST_Y)
        out_ref[...] = jnp.where(mask, s, acc)

    @pl.when(b != LAST_Q // QB)
    def _():
        out_ref[...] = acc


def _project(fmain, T, lastrow, W):
    return pl.pallas_call(
        _project_body,
        grid=(NBLK,),
        in_specs=[
            pl.BlockSpec((FB, 128), lambda i: (i, 0)),
            pl.BlockSpec((16, 128, 128), lambda i: (0, 0, 0)),
            pl.BlockSpec((1, EMB_DIM), lambda i: (0, 0)),
            pl.BlockSpec((EMB_DIM, 1), lambda i: (0, 0)),
        ],
        out_specs=pl.BlockSpec((QB, 128), lambda i: (i, 0)),
        out_shape=jax.ShapeDtypeStruct((NBLK * QB, 128), jnp.float32),
    )(fmain, T, lastrow, W)


def _pool_body(idx_hbm, tv_hbm, b_hbm, out_hbm, idx_v, vals_v, bias_v, acc_v, sem):
    w = lax.axis_index("s") * NC + lax.axis_index("c")

    pltpu.sync_copy(b_hbm, bias_v)
    bias = bias_v[...]

    zero = jnp.zeros((16,), jnp.float32)
    for g in range(RPW // 16):
        acc_v[pl.ds(16 * g, 16)] = zero

    for c in range(N_CHUNKS):
        pltpu.sync_copy(idx_hbm.at[w, pl.ds(c * L_CHUNK * RPW, L_CHUNK * RPW)],
                        idx_v)
        pltpu.async_copy(tv_hbm.at[idx_v], vals_v, sem).wait()
        for g in range(RPW // 16):
            def body(l, a):
                return a + vals_v[pl.ds(l * RPW + 16 * g, 16)]
            part = lax.fori_loop(0, L_CHUNK, body, zero)
            acc_v[pl.ds(16 * g, 16)] += part

    for g in range(RPW // 16):
        acc_v[pl.ds(16 * g, 16)] += bias
    pltpu.sync_copy(acc_v, out_hbm.at[pl.ds(w * RPW, RPW)])


def _pool(idx3d, tv_flat, b16):
    mesh = plsc.VectorSubcoreMesh(core_axis_name="c", subcore_axis_name="s")
    f = pl.kernel(
        _pool_body,
        out_type=jax.ShapeDtypeStruct((BATCH,), jnp.float32),
        mesh=mesh,
        scratch_types=[
            pltpu.VMEM((L_CHUNK * RPW,), jnp.int32),
            pltpu.VMEM((L_CHUNK * RPW,), jnp.float32),
            pltpu.VMEM((16,), jnp.float32),
            pltpu.VMEM((RPW,), jnp.float32),
            pltpu.SemaphoreType.DMA,
        ],
    )
    return f(idx3d, tv_flat, b16)


def kernel(indices, table, W, b):
    Wf = W.astype(jnp.float32)
    tflat = table.reshape(NUM_EMB * EMB_DIM)
    fmain = tflat[: F_ROWS * 128].reshape(F_ROWS, 128)
    lastrow = lax.slice(table, (NUM_EMB - 1, 0), (NUM_EMB, EMB_DIM))
    xs = jnp.arange(128)
    ys = jnp.arange(128)
    js = jnp.arange(16)
    sel = ((ys[None, None, :] // 8 == js[:, None, None])
           & (xs[None, :, None] // 16 == ys[None, None, :] % 8))
    T = jnp.where(sel, (Wf.reshape(-1) * (1.0 / SEQ))[xs % 16][None, :, None],
                  0.0).astype(jnp.float32)
    tv2d = _project(fmain, T, lastrow, Wf)
    tv_flat = tv2d.reshape(NBLK * QB * 128)
    idx = indices.astype(jnp.int32)
    idx3d = (idx.reshape(NW, RPW, SEQ)
                .transpose(0, 2, 1)
                .reshape(NW, SEQ * RPW))
    b16 = jnp.broadcast_to(b.astype(jnp.float32), (16,))
    out = _pool(idx3d, tv_flat, b16)
    return out.reshape(BATCH, 1)


# speedup vs baseline: 8.9853x; 1.0169x over previous
import functools

import jax
import jax.numpy as jnp
from jax import lax
from jax.experimental import pallas as pl
from jax.experimental.pallas import tpu as pltpu
from jax.experimental.pallas import tpu_sc as plsc

NUM_EMB = 1000001
EMB_DIM = 16
BATCH = 16384
SEQ = 200

NC = 2
NS = 16
NW = NC * NS
RPW = BATCH // NW
C_ROWS = 128
N_CHUNKS = RPW // C_ROWS
C_IDX = C_ROWS * SEQ


F_ROWS = 125000
QB = 608
NBLK = 13
FB = 16 * QB
LAST_Q, LAST_Y = divmod(1000000, 128)


def _project_body(f_ref, t_ref, last_ref, w_ref, out_ref):
    b = pl.program_id(0)
    f = f_ref[...]
    frows = lax.broadcasted_iota(jnp.int32, (FB, 128), 0)
    f = jnp.where(frows < F_ROWS - FB * b, f, 0.0)
    g = f.reshape(QB, 16, 128)
    acc = jnp.zeros((QB, 128), jnp.float32)
    for j in range(16):
        acc += jnp.dot(g[:, j, :], t_ref[j],
                       preferred_element_type=jnp.float32)

    @pl.when(b == LAST_Q // QB)
    def _():
        s = jnp.dot(last_ref[...], w_ref[...],
                    preferred_element_type=jnp.float32)[0, 0] * (1.0 / SEQ)
        rows = lax.broadcasted_iota(jnp.int32, (QB, 128), 0)
        cols = lax.broadcasted_iota(jnp.int32, (QB, 128), 1)
        mask = (rows == LAST_Q % QB) & (cols == LAST_Y)
        out_ref[...] = jnp.where(mask, s, acc)

    @pl.when(b != LAST_Q // QB)
    def _():
        out_ref[...] = acc


def _project(fmain, T, lastrow, W):
    return pl.pallas_call(
        _project_body,
        grid=(NBLK,),
        in_specs=[
            pl.BlockSpec((FB, 128), lambda i: (i, 0)),
            pl.BlockSpec((16, 128, 128), lambda i: (0, 0, 0)),
            pl.BlockSpec((1, EMB_DIM), lambda i: (0, 0)),
            pl.BlockSpec((EMB_DIM, 1), lambda i: (0, 0)),
        ],
        out_specs=pl.BlockSpec((QB, 128), lambda i: (i, 0)),
        out_shape=jax.ShapeDtypeStruct((NBLK * QB, 128), jnp.float32),
    )(fmain, T, lastrow, W)


def _pool_body(idx_hbm, tv_hbm, b_hbm, out_hbm,
               idx_v0, idx_v1, vals_v0, vals_v1, bias_v, acc_v, sem0, sem1):
    w = lax.axis_index("s") * NC + lax.axis_index("c")
    base = w * RPW * SEQ
    idx_bufs = (idx_v0, idx_v1)
    val_bufs = (vals_v0, vals_v1)
    sem_bufs = (sem0, sem1)

    pltpu.sync_copy(b_hbm, bias_v)
    bias = bias_v[...]
    zero = jnp.zeros((16,), jnp.float32)
    lane = jax.lax.iota(jnp.int32, 16) * SEQ

    def start(c, buf):
        pltpu.sync_copy(idx_hbm.at[pl.ds(base + c * C_IDX, C_IDX)],
                        idx_bufs[buf])
        return pltpu.async_copy(tv_hbm.at[idx_bufs[buf]], val_bufs[buf],
                                sem_bufs[buf])

    descs = {0: start(0, 0)}
    for c in range(N_CHUNKS):
        buf = c % 2
        if c + 1 < N_CHUNKS:
            descs[c + 1] = start(c + 1, (c + 1) % 2)
        descs[c].wait()
        vb = val_bufs[buf]
        for g in range(C_ROWS // 16):
            gbase = lane + g * 16 * SEQ

            def body(l, a):
                return a + plsc.load_gather(vb, [gbase + l])

            part = lax.fori_loop(0, SEQ, body, zero)
            acc_v[pl.ds(c * C_ROWS + 16 * g, 16)] = part + bias

    pltpu.sync_copy(acc_v, out_hbm.at[pl.ds(w * RPW, RPW)])


def _pool(idxf, tv_flat, b16):
    mesh = plsc.VectorSubcoreMesh(core_axis_name="c", subcore_axis_name="s")
    f = pl.kernel(
        _pool_body,
        out_type=jax.ShapeDtypeStruct((BATCH,), jnp.float32),
        mesh=mesh,
        scratch_types=[
            pltpu.VMEM((C_IDX,), jnp.int32),
            pltpu.VMEM((C_IDX,), jnp.int32),
            pltpu.VMEM((C_IDX,), jnp.float32),
            pltpu.VMEM((C_IDX,), jnp.float32),
            pltpu.VMEM((16,), jnp.float32),
            pltpu.VMEM((RPW,), jnp.float32),
            pltpu.SemaphoreType.DMA,
            pltpu.SemaphoreType.DMA,
        ],
        compiler_params=pltpu.CompilerParams(needs_layout_passes=False),
    )
    return f(idxf, tv_flat, b16)


def kernel(indices, table, W, b):
    Wf = W.astype(jnp.float32)
    tflat = table.reshape(NUM_EMB * EMB_DIM)
    fmain = tflat[: F_ROWS * 128].reshape(F_ROWS, 128)
    lastrow = lax.slice(table, (NUM_EMB - 1, 0), (NUM_EMB, EMB_DIM))
    xs = jnp.arange(128)
    ys = jnp.arange(128)
    js = jnp.arange(16)
    sel = ((ys[None, None, :] // 8 == js[:, None, None])
           & (xs[None, :, None] // 16 == ys[None, None, :] % 8))
    T = jnp.where(sel, (Wf.reshape(-1) * (1.0 / SEQ))[xs % 16][None, :, None],
                  0.0).astype(jnp.float32)
    tv2d = _project(fmain, T, lastrow, Wf)
    tv_flat = tv2d.reshape(NBLK * QB * 128)
    idxf = indices.astype(jnp.int32).reshape(BATCH * SEQ)
    b16 = jnp.broadcast_to(b.astype(jnp.float32), (16,))
    out = _pool(idxf, tv_flat, b16)
    return out.reshape(BATCH, 1)


# speedup vs baseline: 26.3656x; 2.9343x over previous
import jax
import jax.numpy as jnp
from jax import lax
from jax.experimental import pallas as pl
from jax.experimental.pallas import tpu as pltpu
from jax.experimental.pallas import tpu_sc as plsc

NUM_EMB = 1000001
EMB_DIM = 16
BATCH = 16384
SEQ = 200

NC = 2
NS = 16
NW = NC * NS
RPW = BATCH // NW
C_L = 40
N_CHUNKS = SEQ // C_L

CB = 65536
G1 = 16


def _project_body(t_ref, w_ref, out_ref):
    out_ref[...] = jnp.sum(t_ref[...] * w_ref[...], axis=0) * (1.0 / SEQ)


def _project(tableT, W):
    return pl.pallas_call(
        _project_body,
        grid=(G1,),
        in_specs=[
            pl.BlockSpec((EMB_DIM, CB), lambda i: (0, i)),
            pl.BlockSpec((EMB_DIM, 1), lambda i: (0, 0)),
        ],
        out_specs=pl.BlockSpec((CB,), lambda i: (i,)),
        out_shape=jax.ShapeDtypeStruct((G1 * CB,), jnp.float32),
    )(tableT, W)


C_IDX = C_L * RPW


def _pool_body(idx_hbm, tv_hbm, b_hbm, out_hbm,
               idx_v0, idx_v1, vals_v0, vals_v1, bias_v, acc_v, sem0, sem1):
    w = lax.axis_index("s") * NC + lax.axis_index("c")
    col0 = w * RPW
    idx_bufs = (idx_v0, idx_v1)
    val_bufs = (vals_v0, vals_v1)
    sem_bufs = (sem0, sem1)

    pltpu.sync_copy(b_hbm, bias_v)
    bias = bias_v[...]
    zero = jnp.zeros((16,), jnp.float32)
    for g in range(RPW // 16):
        acc_v[pl.ds(16 * g, 16)] = zero

    def start(c, buf):
        for l in range(C_L):
            pltpu.sync_copy(
                idx_hbm.at[pl.ds((c * C_L + l) * BATCH + col0, RPW)],
                idx_bufs[buf].at[pl.ds(l * RPW, RPW)])
        return pltpu.async_copy(tv_hbm.at[idx_bufs[buf]], val_bufs[buf],
                                sem_bufs[buf])

    descs = {0: start(0, 0)}
    for c in range(N_CHUNKS):
        buf = c % 2
        if c + 1 < N_CHUNKS:
            descs[c + 1] = start(c + 1, (c + 1) % 2)
        descs[c].wait()
        vb = val_bufs[buf]
        for g in range(RPW // 16):
            off = 16 * g

            def body(l, a):
                return a + vb[pl.ds(l * RPW + off, 16)]

            part = lax.fori_loop(0, C_L, body, zero)
            acc_v[pl.ds(off, 16)] += part

    for g in range(RPW // 16):
        acc_v[pl.ds(16 * g, 16)] += bias
    pltpu.sync_copy(acc_v, out_hbm.at[pl.ds(col0, RPW)])


def _pool(idxTf, tv_flat, b16):
    mesh = plsc.VectorSubcoreMesh(core_axis_name="c", subcore_axis_name="s")
    f = pl.kernel(
        _pool_body,
        out_type=jax.ShapeDtypeStruct((BATCH,), jnp.float32),
        mesh=mesh,
        scratch_types=[
            pltpu.VMEM((C_IDX,), jnp.int32),
            pltpu.VMEM((C_IDX,), jnp.int32),
            pltpu.VMEM((C_IDX,), jnp.float32),
            pltpu.VMEM((C_IDX,), jnp.float32),
            pltpu.VMEM((16,), jnp.float32),
            pltpu.VMEM((RPW,), jnp.float32),
            pltpu.SemaphoreType.DMA,
            pltpu.SemaphoreType.DMA,
        ],
        compiler_params=pltpu.CompilerParams(needs_layout_passes=False),
    )
    return f(idxTf, tv_flat, b16)


def kernel(indices, table, W, b):
    tableT = table.T
    tv_flat = _project(tableT, W.astype(jnp.float32))
    idxTf = indices.astype(jnp.int32).T.reshape(SEQ * BATCH)
    b16 = jnp.broadcast_to(b.astype(jnp.float32), (16,))
    out = _pool(idxTf, tv_flat, b16)
    return out.reshape(BATCH, 1)


# speedup vs baseline: 31.5700x; 1.1974x over previous
import jax
import jax.numpy as jnp
from jax import lax
from jax.experimental import pallas as pl
from jax.experimental.pallas import tpu as pltpu
from jax.experimental.pallas import tpu_sc as plsc

NUM_EMB = 1000001
EMB_DIM = 16
BATCH = 16384
SEQ = 200

NC = 2
NS = 16
NW = NC * NS
RPW = BATCH // NW
C_L = 40
N_CHUNKS = SEQ // C_L

CB = 65536
G1 = 16


def _project_body(t_ref, w_ref, out_ref):
    out_ref[...] = jnp.sum(t_ref[...] * w_ref[...], axis=0) * (1.0 / SEQ)


def _project(tableT, W):
    return pl.pallas_call(
        _project_body,
        grid=(G1,),
        in_specs=[
            pl.BlockSpec((EMB_DIM, CB), lambda i: (0, i)),
            pl.BlockSpec((EMB_DIM, 1), lambda i: (0, 0)),
        ],
        out_specs=pl.BlockSpec((CB,), lambda i: (i,)),
        out_shape=jax.ShapeDtypeStruct((G1 * CB,), jnp.float32),
    )(tableT, W)


C_IDX = C_L * RPW


def _pool_body(idx_hbm, tv_hbm, b_hbm, out_hbm,
               idx_v0, idx_v1, vals_v0, vals_v1, bias_v, acc_v,
               sem0, sem1, isem0, isem1):
    sid = lax.axis_index("s")
    w = sid * NC + lax.axis_index("c")
    col0 = w * RPW
    idx_bufs = (idx_v0, idx_v1)
    val_bufs = (vals_v0, vals_v1)
    sem_bufs = (sem0, sem1)
    isem_bufs = (isem0, isem1)

    pltpu.sync_copy(b_hbm, bias_v)
    bias = bias_v[...]
    zero = jnp.zeros((16,), jnp.float32)

    def zbody(g, x):
        acc_v[pl.ds(16 * g, 16)] = zero
        return x

    lax.fori_loop(0, RPW // 16, zbody, 0)

    def issue_idx(c, buf):
        base = c * C_L * BATCH + col0

        def body(l, x):
            pltpu.async_copy(
                idx_hbm.at[pl.ds(base + l * BATCH, RPW)],
                idx_bufs[buf].at[pl.ds(l * RPW, RPW)],
                isem_bufs[buf])
            return x

        lax.fori_loop(0, C_L, body, 0)

    def drain_idx(buf):
        pltpu.make_async_copy(idx_hbm.at[pl.ds(0, C_IDX)], idx_bufs[buf],
                              isem_bufs[buf]).wait()

    def gather(buf):
        return pltpu.async_copy(tv_hbm.at[idx_bufs[buf]], val_bufs[buf],
                                sem_bufs[buf])

    issue_idx(0, 0)
    drain_idx(0)
    gdescs = {0: gather(0)}
    for c in range(N_CHUNKS):
        buf = c % 2
        if c + 1 < N_CHUNKS:
            nbuf = (c + 1) % 2
            issue_idx(c + 1, nbuf)
            drain_idx(nbuf)
            gdescs[c + 1] = gather(nbuf)
        gdescs[c].wait()
        vb = val_bufs[buf]

        def gbody(g, x):
            off = 16 * g

            def body(j, a):
                p = j * 5 * RPW + off
                for r in range(5):
                    a = a + vb[pl.ds(p + r * RPW, 16)]
                return a

            part = lax.fori_loop(0, C_L // 5, body, zero)
            acc_v[pl.ds(off, 16)] += part
            return x

        lax.fori_loop(0, RPW // 16, gbody, 0)

    def bbody(g, x):
        acc_v[pl.ds(16 * g, 16)] += bias
        return x

    lax.fori_loop(0, RPW // 16, bbody, 0)
    pltpu.sync_copy(acc_v, out_hbm.at[pl.ds(col0, RPW)])


def _pool(idxTf, tv_flat, b16):
    mesh = plsc.VectorSubcoreMesh(core_axis_name="c", subcore_axis_name="s")
    f = pl.kernel(
        _pool_body,
        out_type=jax.ShapeDtypeStruct((BATCH,), jnp.float32),
        mesh=mesh,
        scratch_types=[
            pltpu.VMEM((C_IDX,), jnp.int32),
            pltpu.VMEM((C_IDX,), jnp.int32),
            pltpu.VMEM((C_IDX,), jnp.float32),
            pltpu.VMEM((C_IDX,), jnp.float32),
            pltpu.VMEM((16,), jnp.float32),
            pltpu.VMEM((RPW,), jnp.float32),
            pltpu.SemaphoreType.DMA,
            pltpu.SemaphoreType.DMA,
            pltpu.SemaphoreType.DMA,
            pltpu.SemaphoreType.DMA,
        ],
        compiler_params=pltpu.CompilerParams(needs_layout_passes=False),
    )
    return f(idxTf, tv_flat, b16)


def kernel(indices, table, W, b):
    tableT = table.T
    tv_flat = _project(tableT, W.astype(jnp.float32))
    idxTf = indices.astype(jnp.int32).T.reshape(SEQ * BATCH)
    b16 = jnp.broadcast_to(b.astype(jnp.float32), (16,))
    out = _pool(idxTf, tv_flat, b16)
    return out.reshape(BATCH, 1)
